# Initial kernel scaffold; baseline (speedup 1.0000x reference)
import jax
import jax.numpy as jnp
from jax.experimental import pallas as pl


def kernel(inputs, params):
    raise NotImplementedError("write your pallas kernel here")


# speedup vs baseline: 10.9335x; 10.9335x over previous
import functools

import jax
import jax.numpy as jnp
from jax import lax
from jax.experimental import pallas as pl
from jax.experimental.pallas import tpu as pltpu
from jax.experimental.pallas import tpu_sc as plsc

_SA_CFG = [(4096, 0.1, 32, (32, 64)), (1024, 0.2, 32, (64, 128)),
           (256, 0.4, 32, (128, 256)), (64, 0.8, 32, (256, 512))]


def _fps_body(x_ref, oc_ref, *, n, npoint):
    xs = x_ref[0, 0]
    ys = x_ref[0, 1]
    zs = x_ref[0, 2]
    r = n // 128
    pid = (lax.broadcasted_iota(jnp.int32, (r, 128), 0) * 128
           + lax.broadcasted_iota(jnp.int32, (r, 128), 1))
    opid = lax.broadcasted_iota(jnp.int32, (1, npoint), 1)

    def step(i, carry):
        dists, far, ocx, ocy, ocz = carry
        sel = pid == far
        cx = jnp.sum(jnp.where(sel, xs, 0.0))
        cy = jnp.sum(jnp.where(sel, ys, 0.0))
        cz = jnp.sum(jnp.where(sel, zs, 0.0))
        rec = opid == i
        ocx = jnp.where(rec, cx, ocx)
        ocy = jnp.where(rec, cy, ocy)
        ocz = jnp.where(rec, cz, ocz)
        d = (xs - cx) ** 2 + (ys - cy) ** 2 + (zs - cz) ** 2
        dists = jnp.minimum(dists, d)
        mx = jnp.max(dists)
        nxt = jnp.min(jnp.where(dists == mx, pid, n))
        return dists, nxt, ocx, ocy, ocz

    init = (jnp.full((r, 128), 1e10, jnp.float32), jnp.int32(0),
            jnp.zeros((1, npoint), jnp.float32),
            jnp.zeros((1, npoint), jnp.float32),
            jnp.zeros((1, npoint), jnp.float32))
    _, _, ocx, ocy, ocz = lax.fori_loop(0, npoint, step, init)
    oc_ref[0, 0:1, :] = ocx
    oc_ref[0, 1:2, :] = ocy
    oc_ref[0, 2:3, :] = ocz


def _fps(xyz_t, npoint):
    b, _, n = xyz_t.shape
    x4 = xyz_t.reshape(b, 3, n // 128, 128)
    return pl.pallas_call(
        functools.partial(_fps_body, n=n, npoint=npoint),
        grid=(b,),
        in_specs=[pl.BlockSpec((1, 3, n // 128, 128), lambda i: (i, 0, 0, 0))],
        out_specs=pl.BlockSpec((1, 3, npoint), lambda i: (i, 0, 0)),
        out_shape=jax.ShapeDtypeStruct((b, 3, npoint), jnp.float32),
    )(x4)


def _bf16_dot3(x1, y1, z1, x2, y2, z2):
    r = lambda v: v.astype(jnp.bfloat16).astype(jnp.float32)
    return r(x1) * r(x2) + r(y1) * r(y2) + r(z1) * r(z2)


def _ballq_body(x_ref, c_ref, o_ref, *, n, mq, nsample, r2):
    x2 = x_ref[0, 0:1, :]
    y2 = x_ref[0, 1:2, :]
    z2 = x_ref[0, 2:3, :]
    c = c_ref[0]
    x1 = c[:, 0:1]
    y1 = c[:, 1:2]
    z1 = c[:, 2:3]
    aa = x1 * x1 + y1 * y1 + z1 * z1
    bb = x2 * x2 + y2 * y2 + z2 * z2
    ab = _bf16_dot3(x1, y1, z1, x2, y2, z2)
    d = aa + bb - 2.0 * ab
    pid = lax.broadcasted_iota(jnp.int32, (mq, n), 1)
    cand = jnp.where(d <= r2, pid, n)
    lane = lax.broadcasted_iota(jnp.int32, (mq, nsample), 1)
    out = jnp.full((mq, nsample), n, jnp.int32)
    for k in range(nsample):
        sel = jnp.min(cand, axis=1, keepdims=True)
        out = jnp.where(lane == k, sel, out)
        cand = jnp.where(cand == sel, n, cand)
    first = out[:, 0:1]
    out = jnp.where(out == n, first, out)
    o_ref[0] = jnp.clip(out, 0, n - 1)


def _ballq(xyz_t, new_xyz, radius, nsample):
    b, _, n = xyz_t.shape
    npt = new_xyz.shape[1]
    mq = min(128, npt)
    return pl.pallas_call(
        functools.partial(_ballq_body, n=n, mq=mq, nsample=nsample,
                          r2=float(radius) ** 2),
        grid=(b, npt // mq),
        in_specs=[pl.BlockSpec((1, 3, n), lambda i, j: (i, 0, 0)),
                  pl.BlockSpec((1, mq, 3), lambda i, j: (i, j, 0))],
        out_specs=pl.BlockSpec((1, mq, nsample), lambda i, j: (i, j, 0)),
        out_shape=jax.ShapeDtypeStruct((b, npt, nsample), jnp.int32),
    )(xyz_t, new_xyz)


def _mm_body(x_ref, w_ref, b_ref, o_ref, *, relu):
    acc = jnp.dot(x_ref[...], w_ref[...],
                  preferred_element_type=jnp.float32) + b_ref[...]
    o_ref[...] = jnp.maximum(acc, 0.0) if relu else acc


def _mm(x, wt, bias, relu):
    rr, k = x.shape
    o = wt.shape[1]
    tr = min(512, rr)
    return pl.pallas_call(
        functools.partial(_mm_body, relu=relu),
        grid=(rr // tr,),
        in_specs=[pl.BlockSpec((tr, k), lambda i: (i, 0)),
                  pl.BlockSpec((k, o), lambda i: (0, 0)),
                  pl.BlockSpec((1, o), lambda i: (0, 0))],
        out_specs=pl.BlockSpec((tr, o), lambda i: (i, 0)),
        out_shape=jax.ShapeDtypeStruct((rr, o), jnp.float32),
    )(x, wt, bias.reshape(1, o))


def _sc_gather(table, idx):
    v, c = table.shape
    btot = idx.shape[0]
    info = plsc.get_sparse_core_info()
    nc, ns = info.num_cores, info.num_subcores
    nw = nc * ns
    b_per_w = btot // nw
    chunk = b_per_w
    while chunk * (c + 1) * 4 > 380_000:
        chunk //= 2
    n_chunks = b_per_w // chunk
    mesh = plsc.VectorSubcoreMesh(core_axis_name="c", subcore_axis_name="s")

    @functools.partial(
        pl.kernel, mesh=mesh,
        out_type=jax.ShapeDtypeStruct((btot, c), jnp.float32),
        scratch_types=[
            pltpu.VMEM((chunk,), jnp.int32),
            pltpu.VMEM((chunk, c), jnp.float32),
            pltpu.SemaphoreType.DMA,
        ],
    )
    def gk(table_hbm, idx_hbm, out_hbm, idx_v, rows_v, sem):
        wid = lax.axis_index("s") * nc + lax.axis_index("c")
        base = wid * b_per_w
        for ci in range(n_chunks):
            off = base + ci * chunk
            pltpu.sync_copy(idx_hbm.at[pl.ds(off, chunk)], idx_v)
            pltpu.async_copy(table_hbm.at[idx_v], rows_v, sem).wait()
            pltpu.sync_copy(rows_v, out_hbm.at[pl.ds(off, chunk)])

    return gk(table, idx)


def _sa2_body(g_ref, c_ref, wa_ref, w2_ref, b2_ref, o_ref, *, mg, ns, c1, c2):
    q = jnp.dot(c_ref[...], wa_ref[...], preferred_element_type=jnp.float32)
    g = g_ref[...].reshape(mg, ns, c1)
    h = jnp.maximum(g - q[:, None, :], 0.0)
    h2 = jnp.dot(h.reshape(mg * ns, c1), w2_ref[...],
                 preferred_element_type=jnp.float32) + b2_ref[...]
    h2 = jnp.maximum(h2, 0.0)
    o_ref[...] = jnp.max(h2.reshape(mg, ns, c2), axis=1)


def _sa2(grows, newc, wa_t, w2_t, b2, ns):
    rtot = newc.shape[0]
    c1 = grows.shape[1]
    c2 = w2_t.shape[1]
    mg = min(64, rtot)
    return pl.pallas_call(
        functools.partial(_sa2_body, mg=mg, ns=ns, c1=c1, c2=c2),
        grid=(rtot // mg,),
        in_specs=[pl.BlockSpec((mg * ns, c1), lambda i: (i, 0)),
                  pl.BlockSpec((mg, 3), lambda i: (i, 0)),
                  pl.BlockSpec((3, c1), lambda i: (0, 0)),
                  pl.BlockSpec((c1, c2), lambda i: (0, 0)),
                  pl.BlockSpec((1, c2), lambda i: (0, 0))],
        out_specs=pl.BlockSpec((mg, c2), lambda i: (i, 0)),
        out_shape=jax.ShapeDtypeStruct((rtot, c2), jnp.float32),
    )(grows, newc, wa_t, w2_t, b2.reshape(1, c2))


def _top3_body(c_ref, x_ref, oi_ref, ow_ref, *, n2, mf):
    c = c_ref[0]
    x1 = c[:, 0:1]
    y1 = c[:, 1:2]
    z1 = c[:, 2:3]
    x2 = x_ref[0, 0:1, :]
    y2 = x_ref[0, 1:2, :]
    z2 = x_ref[0, 2:3, :]
    aa = x1 * x1 + y1 * y1 + z1 * z1
    bb = x2 * x2 + y2 * y2 + z2 * z2
    ab = _bf16_dot3(x1, y1, z1, x2, y2, z2)
    d = aa + bb - 2.0 * ab
    pid = lax.broadcasted_iota(jnp.int32, (mf, n2), 1)
    lane = lax.broadcasted_iota(jnp.int32, (mf, 3), 1)
    oi = jnp.zeros((mf, 3), jnp.int32)
    ov = jnp.zeros((mf, 3), jnp.float32)
    for k in range(3):
        mn = jnp.min(d, axis=1, keepdims=True)
        ik = jnp.min(jnp.where(d == mn, pid, n2), axis=1, keepdims=True)
        oi = jnp.where(lane == k, ik, oi)
        ov = jnp.where(lane == k, mn, ov)
        d = jnp.where(pid == ik, 1e30, d)
    dist = jnp.maximum(ov, 1e-10)
    recip = 1.0 / dist
    w = recip / jnp.sum(recip, axis=1, keepdims=True)
    oi_ref[0] = oi
    ow_ref[0] = w


def _top3(xyz1, xyz2_t):
    b, n1, _ = xyz1.shape
    n2 = xyz2_t.shape[2]
    mf = min(256, n1)
    return pl.pallas_call(
        functools.partial(_top3_body, n2=n2, mf=mf),
        grid=(b, n1 // mf),
        in_specs=[pl.BlockSpec((1, mf, 3), lambda i, j: (i, j, 0)),
                  pl.BlockSpec((1, 3, n2), lambda i, j: (i, 0, 0))],
        out_specs=[pl.BlockSpec((1, mf, 3), lambda i, j: (i, j, 0)),
                   pl.BlockSpec((1, mf, 3), lambda i, j: (i, j, 0))],
        out_shape=[jax.ShapeDtypeStruct((b, n1, 3), jnp.int32),
                   jax.ShapeDtypeStruct((b, n1, 3), jnp.float32)],
    )(xyz1, xyz2_t)


def _fpmlp_body(r_ref, w3_ref, s_ref, ws_ref, wi_ref, b1_ref, w2_ref, b2_ref,
                o_ref, *, mr, c2):
    r3 = r_ref[...].reshape(mr, 3, c2)
    w3 = w3_ref[...]
    interp = jnp.sum(r3 * w3[:, :, None], axis=1)
    h = jnp.dot(s_ref[...], ws_ref[...], preferred_element_type=jnp.float32)
    h = h + jnp.dot(interp, wi_ref[...], preferred_element_type=jnp.float32)
    h = jnp.maximum(h + b1_ref[...], 0.0)
    o = jnp.dot(h, w2_ref[...], preferred_element_type=jnp.float32) + b2_ref[...]
    o_ref[...] = jnp.maximum(o, 0.0)


def _fpmlp(rows, w3, skip, ws_t, wi_t, b1, w2_t, b2):
    rtot = w3.shape[0]
    c2 = rows.shape[1]
    cs = skip.shape[1]
    o1 = ws_t.shape[1]
    o2 = w2_t.shape[1]
    mr = min(256, rtot)
    return pl.pallas_call(
        functools.partial(_fpmlp_body, mr=mr, c2=c2),
        grid=(rtot // mr,),
        in_specs=[pl.BlockSpec((mr * 3, c2), lambda i: (i, 0)),
                  pl.BlockSpec((mr, 3), lambda i: (i, 0)),
                  pl.BlockSpec((mr, cs), lambda i: (i, 0)),
                  pl.BlockSpec((cs, o1), lambda i: (0, 0)),
                  pl.BlockSpec((c2, o1), lambda i: (0, 0)),
                  pl.BlockSpec((1, o1), lambda i: (0, 0)),
                  pl.BlockSpec((o1, o2), lambda i: (0, 0)),
                  pl.BlockSpec((1, o2), lambda i: (0, 0))],
        out_specs=pl.BlockSpec((mr, o2), lambda i: (i, 0)),
        out_shape=jax.ShapeDtypeStruct((rtot, o2), jnp.float32),
    )(rows, w3, skip, ws_t, wi_t, b1.reshape(1, o1), w2_t, b2.reshape(1, o2))


def kernel(inputs, params):
    b, n0, _ = inputs.shape
    coords = inputs[..., :3]
    feats = inputs
    xyz = coords
    xyz_t = jnp.transpose(coords, (0, 2, 1))
    coords_list, feats_list = [], []
    for p, (npoint, radius, nsample, _) in zip(params['sa'], _SA_CFG):
        coords_list.append(xyz)
        feats_list.append(feats)
        n = xyz.shape[1]
        cf = feats.shape[2]
        newc_t = _fps(xyz_t, npoint)
        new_xyz = jnp.transpose(newc_t, (0, 2, 1))
        gi = _ballq(xyz_t, new_xyz, radius, nsample)
        (w1, b1), (w2, b2) = p
        c1 = w1.shape[0]
        c1p = -(-c1 // 128) * 128
        xin = jnp.concatenate([xyz, feats], -1).reshape(b * n, 3 + cf)
        w1t = jnp.pad(w1.T, ((0, 0), (0, c1p - c1)))
        g1 = _mm(xin, w1t, jnp.pad(b1, (0, c1p - c1)), relu=False)
        flat_gi = (gi + (jnp.arange(b, dtype=jnp.int32) * n)[:, None, None]
                   ).reshape(b * npoint * nsample)
        grows = _sc_gather(g1, flat_gi)
        wa_t = jnp.pad(w1[:, :3].T, ((0, 0), (0, c1p - c1)))
        w2t = jnp.pad(w2.T, ((0, c1p - c1), (0, 0)))
        feats = _sa2(grows, new_xyz.reshape(b * npoint, 3), wa_t, w2t, b2,
                     nsample).reshape(b, npoint, -1)
        xyz, xyz_t = new_xyz, newc_t
    feats_list[0] = inputs[..., 3:]
    for i, p in enumerate(params['fp']):
        xyz1 = coords_list[-1 - i]
        skip = feats_list[-1 - i]
        n1 = xyz1.shape[1]
        n2 = xyz.shape[1]
        c2 = feats.shape[2]
        cs = skip.shape[2]
        idx3, w3 = _top3(xyz1, xyz_t)
        flat = (idx3 + (jnp.arange(b, dtype=jnp.int32) * n2)[:, None, None]
                ).reshape(b * n1 * 3)
        rows = _sc_gather(feats.reshape(b * n2, c2), flat)
        (w1, b1), (w2, b2) = p
        feats = _fpmlp(rows, w3.reshape(b * n1, 3), skip.reshape(b * n1, cs),
                       w1[:, :cs].T, w1[:, cs:].T, b1, w2.T, b2
                       ).reshape(b, n1, -1)
        xyz, xyz_t = xyz1, jnp.transpose(xyz1, (0, 2, 1))
    (wc1, bc1), (wc2, bc2) = params['cls']
    c = feats.shape[2]
    h = _mm(feats.reshape(b * n0, c), wc1.T, bc1, relu=True)
    out = _mm(h, wc2.T, bc2, relu=False)
    return out.reshape(b, n0, -1)


# speedup vs baseline: 11.1023x; 1.0154x over previous
import functools

import jax
import jax.numpy as jnp
from jax import lax
from jax.experimental import pallas as pl
from jax.experimental.pallas import tpu as pltpu
from jax.experimental.pallas import tpu_sc as plsc

_SA_CFG = [(4096, 0.1, 32, (32, 64)), (1024, 0.2, 32, (64, 128)),
           (256, 0.4, 32, (128, 256)), (64, 0.8, 32, (256, 512))]


def _fps_body(x_ref, oc_ref, *, n, npoint):
    xs = x_ref[0, 0]
    ys = x_ref[0, 1]
    zs = x_ref[0, 2]
    r = n // 128
    pid = (lax.broadcasted_iota(jnp.int32, (r, 128), 0) * 128
           + lax.broadcasted_iota(jnp.int32, (r, 128), 1))
    opid = lax.broadcasted_iota(jnp.int32, (1, npoint), 1)

    def step(i, carry):
        dists, far, ocx, ocy, ocz = carry
        sel = pid == far
        cx = jnp.sum(jnp.where(sel, xs, 0.0))
        cy = jnp.sum(jnp.where(sel, ys, 0.0))
        cz = jnp.sum(jnp.where(sel, zs, 0.0))
        rec = opid == i
        ocx = jnp.where(rec, cx, ocx)
        ocy = jnp.where(rec, cy, ocy)
        ocz = jnp.where(rec, cz, ocz)
        d = (xs - cx) ** 2 + (ys - cy) ** 2 + (zs - cz) ** 2
        dists = jnp.minimum(dists, d)
        mx = jnp.max(dists)
        nxt = jnp.min(jnp.where(dists == mx, pid, n))
        return dists, nxt, ocx, ocy, ocz

    init = (jnp.full((r, 128), 1e10, jnp.float32), jnp.int32(0),
            jnp.zeros((1, npoint), jnp.float32),
            jnp.zeros((1, npoint), jnp.float32),
            jnp.zeros((1, npoint), jnp.float32))
    _, _, ocx, ocy, ocz = lax.fori_loop(0, npoint, step, init)
    oc_ref[0, 0:1, :] = ocx
    oc_ref[0, 1:2, :] = ocy
    oc_ref[0, 2:3, :] = ocz


def _fps(xyz_t, npoint):
    b, _, n = xyz_t.shape
    x4 = xyz_t.reshape(b, 3, n // 128, 128)
    return pl.pallas_call(
        functools.partial(_fps_body, n=n, npoint=npoint),
        grid=(b,),
        in_specs=[pl.BlockSpec((1, 3, n // 128, 128), lambda i: (i, 0, 0, 0))],
        out_specs=pl.BlockSpec((1, 3, npoint), lambda i: (i, 0, 0)),
        out_shape=jax.ShapeDtypeStruct((b, 3, npoint), jnp.float32),
    )(x4)


def _bf16_dot3(c, x3):
    return jnp.dot(c.astype(jnp.bfloat16), x3.astype(jnp.bfloat16),
                   preferred_element_type=jnp.float32)


def _ballq_body(x_ref, c_ref, o_ref, *, n, mq, nsample, r2):
    x2 = x_ref[0, 0:1, :]
    y2 = x_ref[0, 1:2, :]
    z2 = x_ref[0, 2:3, :]
    c = c_ref[0]
    x1 = c[:, 0:1]
    y1 = c[:, 1:2]
    z1 = c[:, 2:3]
    aa = x1 * x1 + y1 * y1 + z1 * z1
    bb = x2 * x2 + y2 * y2 + z2 * z2
    ab = _bf16_dot3(c, x_ref[0])
    d = aa + bb - 2.0 * ab
    pid = lax.broadcasted_iota(jnp.int32, (mq, n), 1)
    cand = jnp.where(d <= r2, pid, n)
    lane = lax.broadcasted_iota(jnp.int32, (mq, nsample), 1)
    out = jnp.full((mq, nsample), n, jnp.int32)
    for k in range(nsample):
        sel = jnp.min(cand, axis=1, keepdims=True)
        out = jnp.where(lane == k, sel, out)
        cand = jnp.where(cand == sel, n, cand)
    first = out[:, 0:1]
    out = jnp.where(out == n, first, out)
    o_ref[0] = jnp.clip(out, 0, n - 1)


def _ballq(xyz_t, new_xyz, radius, nsample):
    b, _, n = xyz_t.shape
    npt = new_xyz.shape[1]
    mq = min(128, npt)
    return pl.pallas_call(
        functools.partial(_ballq_body, n=n, mq=mq, nsample=nsample,
                          r2=float(radius) ** 2),
        grid=(b, npt // mq),
        in_specs=[pl.BlockSpec((1, 3, n), lambda i, j: (i, 0, 0)),
                  pl.BlockSpec((1, mq, 3), lambda i, j: (i, j, 0))],
        out_specs=pl.BlockSpec((1, mq, nsample), lambda i, j: (i, j, 0)),
        out_shape=jax.ShapeDtypeStruct((b, npt, nsample), jnp.int32),
    )(xyz_t, new_xyz)


def _mm_body(x_ref, w_ref, b_ref, o_ref, *, relu):
    acc = jnp.dot(x_ref[...], w_ref[...],
                  preferred_element_type=jnp.float32) + b_ref[...]
    o_ref[...] = jnp.maximum(acc, 0.0) if relu else acc


def _mm(x, wt, bias, relu):
    rr, k = x.shape
    o = wt.shape[1]
    tr = min(512, rr)
    return pl.pallas_call(
        functools.partial(_mm_body, relu=relu),
        grid=(rr // tr,),
        in_specs=[pl.BlockSpec((tr, k), lambda i: (i, 0)),
                  pl.BlockSpec((k, o), lambda i: (0, 0)),
                  pl.BlockSpec((1, o), lambda i: (0, 0))],
        out_specs=pl.BlockSpec((tr, o), lambda i: (i, 0)),
        out_shape=jax.ShapeDtypeStruct((rr, o), jnp.float32),
    )(x, wt, bias.reshape(1, o))


def _sc_gather(table, idx):
    v, c = table.shape
    btot = idx.shape[0]
    info = plsc.get_sparse_core_info()
    nc, ns = info.num_cores, info.num_subcores
    nw = nc * ns
    b_per_w = btot // nw
    chunk = b_per_w
    while chunk * (c + 1) * 4 > 380_000:
        chunk //= 2
    n_chunks = b_per_w // chunk
    mesh = plsc.VectorSubcoreMesh(core_axis_name="c", subcore_axis_name="s")

    @functools.partial(
        pl.kernel, mesh=mesh,
        out_type=jax.ShapeDtypeStruct((btot, c), jnp.float32),
        scratch_types=[
            pltpu.VMEM((chunk,), jnp.int32),
            pltpu.VMEM((chunk, c), jnp.float32),
            pltpu.SemaphoreType.DMA,
        ],
    )
    def gk(table_hbm, idx_hbm, out_hbm, idx_v, rows_v, sem):
        wid = lax.axis_index("s") * nc + lax.axis_index("c")
        base = wid * b_per_w
        for ci in range(n_chunks):
            off = base + ci * chunk
            pltpu.sync_copy(idx_hbm.at[pl.ds(off, chunk)], idx_v)
            pltpu.async_copy(table_hbm.at[idx_v], rows_v, sem).wait()
            pltpu.sync_copy(rows_v, out_hbm.at[pl.ds(off, chunk)])

    return gk(table, idx)


def _sa2_body(g_ref, c_ref, wa_ref, w2_ref, b2_ref, o_ref, *, mg, ns, c1, c2):
    q = jnp.dot(c_ref[...], wa_ref[...], preferred_element_type=jnp.float32)
    g = g_ref[...].reshape(mg, ns, c1)
    h = jnp.maximum(g - q[:, None, :], 0.0)
    h2 = jnp.dot(h.reshape(mg * ns, c1), w2_ref[...],
                 preferred_element_type=jnp.float32) + b2_ref[...]
    h2 = jnp.maximum(h2, 0.0)
    o_ref[...] = jnp.max(h2.reshape(mg, ns, c2), axis=1)


def _sa2(grows, newc, wa_t, w2_t, b2, ns):
    rtot = newc.shape[0]
    c1 = grows.shape[1]
    c2 = w2_t.shape[1]
    mg = min(64, rtot)
    return pl.pallas_call(
        functools.partial(_sa2_body, mg=mg, ns=ns, c1=c1, c2=c2),
        grid=(rtot // mg,),
        in_specs=[pl.BlockSpec((mg * ns, c1), lambda i: (i, 0)),
                  pl.BlockSpec((mg, 3), lambda i: (i, 0)),
                  pl.BlockSpec((3, c1), lambda i: (0, 0)),
                  pl.BlockSpec((c1, c2), lambda i: (0, 0)),
                  pl.BlockSpec((1, c2), lambda i: (0, 0))],
        out_specs=pl.BlockSpec((mg, c2), lambda i: (i, 0)),
        out_shape=jax.ShapeDtypeStruct((rtot, c2), jnp.float32),
    )(grows, newc, wa_t, w2_t, b2.reshape(1, c2))


def _top3_body(c_ref, x_ref, oi_ref, ow_ref, *, n2, mf):
    c = c_ref[0]
    x1 = c[:, 0:1]
    y1 = c[:, 1:2]
    z1 = c[:, 2:3]
    x2 = x_ref[0, 0:1, :]
    y2 = x_ref[0, 1:2, :]
    z2 = x_ref[0, 2:3, :]
    aa = x1 * x1 + y1 * y1 + z1 * z1
    bb = x2 * x2 + y2 * y2 + z2 * z2
    ab = _bf16_dot3(c, x_ref[0])
    d = aa + bb - 2.0 * ab
    pid = lax.broadcasted_iota(jnp.int32, (mf, n2), 1)
    lane = lax.broadcasted_iota(jnp.int32, (mf, 3), 1)
    oi = jnp.zeros((mf, 3), jnp.int32)
    ov = jnp.zeros((mf, 3), jnp.float32)
    for k in range(3):
        mn = jnp.min(d, axis=1, keepdims=True)
        ik = jnp.min(jnp.where(d == mn, pid, n2), axis=1, keepdims=True)
        oi = jnp.where(lane == k, ik, oi)
        ov = jnp.where(lane == k, mn, ov)
        d = jnp.where(pid == ik, 1e30, d)
    dist = jnp.maximum(ov, 1e-10)
    recip = 1.0 / dist
    w = recip / jnp.sum(recip, axis=1, keepdims=True)
    oi_ref[0] = oi
    ow_ref[0] = w


def _top3(xyz1, xyz2_t):
    b, n1, _ = xyz1.shape
    n2 = xyz2_t.shape[2]
    mf = min(256, n1)
    return pl.pallas_call(
        functools.partial(_top3_body, n2=n2, mf=mf),
        grid=(b, n1 // mf),
        in_specs=[pl.BlockSpec((1, mf, 3), lambda i, j: (i, j, 0)),
                  pl.BlockSpec((1, 3, n2), lambda i, j: (i, 0, 0))],
        out_specs=[pl.BlockSpec((1, mf, 3), lambda i, j: (i, j, 0)),
                   pl.BlockSpec((1, mf, 3), lambda i, j: (i, j, 0))],
        out_shape=[jax.ShapeDtypeStruct((b, n1, 3), jnp.int32),
                   jax.ShapeDtypeStruct((b, n1, 3), jnp.float32)],
    )(xyz1, xyz2_t)


def _fpmlp_body(r_ref, w3_ref, s_ref, ws_ref, wi_ref, b1_ref, w2_ref, b2_ref,
                o_ref, *, mr, c2):
    r3 = r_ref[...].reshape(mr, 3, c2)
    w3 = w3_ref[...]
    interp = jnp.sum(r3 * w3[:, :, None], axis=1)
    h = jnp.dot(s_ref[...], ws_ref[...], preferred_element_type=jnp.float32)
    h = h + jnp.dot(interp, wi_ref[...], preferred_element_type=jnp.float32)
    h = jnp.maximum(h + b1_ref[...], 0.0)
    o = jnp.dot(h, w2_ref[...], preferred_element_type=jnp.float32) + b2_ref[...]
    o_ref[...] = jnp.maximum(o, 0.0)


def _fpmlp(rows, w3, skip, ws_t, wi_t, b1, w2_t, b2):
    rtot = w3.shape[0]
    c2 = rows.shape[1]
    cs = skip.shape[1]
    o1 = ws_t.shape[1]
    o2 = w2_t.shape[1]
    mr = min(256, rtot)
    return pl.pallas_call(
        functools.partial(_fpmlp_body, mr=mr, c2=c2),
        grid=(rtot // mr,),
        in_specs=[pl.BlockSpec((mr * 3, c2), lambda i: (i, 0)),
                  pl.BlockSpec((mr, 3), lambda i: (i, 0)),
                  pl.BlockSpec((mr, cs), lambda i: (i, 0)),
                  pl.BlockSpec((cs, o1), lambda i: (0, 0)),
                  pl.BlockSpec((c2, o1), lambda i: (0, 0)),
                  pl.BlockSpec((1, o1), lambda i: (0, 0)),
                  pl.BlockSpec((o1, o2), lambda i: (0, 0)),
                  pl.BlockSpec((1, o2), lambda i: (0, 0))],
        out_specs=pl.BlockSpec((mr, o2), lambda i: (i, 0)),
        out_shape=jax.ShapeDtypeStruct((rtot, o2), jnp.float32),
    )(rows, w3, skip, ws_t, wi_t, b1.reshape(1, o1), w2_t, b2.reshape(1, o2))


def kernel(inputs, params):
    b, n0, _ = inputs.shape
    coords = inputs[..., :3]
    feats = inputs
    xyz = coords
    xyz_t = jnp.transpose(coords, (0, 2, 1))
    coords_list, feats_list = [], []
    for p, (npoint, radius, nsample, _) in zip(params['sa'], _SA_CFG):
        coords_list.append(xyz)
        feats_list.append(feats)
        n = xyz.shape[1]
        cf = feats.shape[2]
        newc_t = _fps(xyz_t, npoint)
        new_xyz = jnp.transpose(newc_t, (0, 2, 1))
        gi = _ballq(xyz_t, new_xyz, radius, nsample)
        (w1, b1), (w2, b2) = p
        c1 = w1.shape[0]
        c1p = -(-c1 // 128) * 128
        xin = jnp.concatenate([xyz, feats], -1).reshape(b * n, 3 + cf)
        w1t = jnp.pad(w1.T, ((0, 0), (0, c1p - c1)))
        g1 = _mm(xin, w1t, jnp.pad(b1, (0, c1p - c1)), relu=False)
        flat_gi = (gi + (jnp.arange(b, dtype=jnp.int32) * n)[:, None, None]
                   ).reshape(b * npoint * nsample)
        grows = _sc_gather(g1, flat_gi)
        wa_t = jnp.pad(w1[:, :3].T, ((0, 0), (0, c1p - c1)))
        w2t = jnp.pad(w2.T, ((0, c1p - c1), (0, 0)))
        feats = _sa2(grows, new_xyz.reshape(b * npoint, 3), wa_t, w2t, b2,
                     nsample).reshape(b, npoint, -1)
        xyz, xyz_t = new_xyz, newc_t
    feats_list[0] = inputs[..., 3:]
    for i, p in enumerate(params['fp']):
        xyz1 = coords_list[-1 - i]
        skip = feats_list[-1 - i]
        n1 = xyz1.shape[1]
        n2 = xyz.shape[1]
        c2 = feats.shape[2]
        cs = skip.shape[2]
        idx3, w3 = _top3(xyz1, xyz_t)
        flat = (idx3 + (jnp.arange(b, dtype=jnp.int32) * n2)[:, None, None]
                ).reshape(b * n1 * 3)
        rows = _sc_gather(feats.reshape(b * n2, c2), flat)
        (w1, b1), (w2, b2) = p
        feats = _fpmlp(rows, w3.reshape(b * n1, 3), skip.reshape(b * n1, cs),
                       w1[:, :cs].T, w1[:, cs:].T, b1, w2.T, b2
                       ).reshape(b, n1, -1)
        xyz, xyz_t = xyz1, jnp.transpose(xyz1, (0, 2, 1))
    (wc1, bc1), (wc2, bc2) = params['cls']
    c = feats.shape[2]
    h = _mm(feats.reshape(b * n0, c), wc1.T, bc1, relu=True)
    out = _mm(h, wc2.T, bc2, relu=False)
    return out.reshape(b, n0, -1)
